# baseline (speedup 1.0000x reference)
import functools

import jax
import jax.numpy as jnp
from jax import lax
from jax.experimental import pallas as pl
from jax.experimental.pallas import tpu as pltpu
from jax.experimental.pallas import tpu_sc as plsc

DIM = 64
PADW = 128
BATCH = 4096
HIST = 200
VOCAB = 1000000

NC = 2
NS = 16
NW = NC * NS
BBLK = BATCH // NW
L = 16

TBLK = 2048
NTBLK = (VOCAB + TBLK - 1) // TBLK


@functools.partial(
    pl.pallas_call,
    grid=(NTBLK,),
    in_specs=[pl.BlockSpec((DIM, TBLK), lambda j: (0, j))],
    out_specs=pl.BlockSpec((TBLK, PADW), lambda j: (j, 0)),
    out_shape=jax.ShapeDtypeStruct((VOCAB, PADW), jnp.float32),
)
def _transpose_pad(tT_ref, out_ref):
    t = tT_ref[...].T
    out_ref[...] = jnp.concatenate([t, t], axis=1)


@functools.partial(
    pl.kernel,
    mesh=plsc.VectorSubcoreMesh(core_axis_name="c", subcore_axis_name="s"),
    out_type=jax.ShapeDtypeStruct((HIST, DIM, BATCH), jnp.float32),
    scratch_types=[
        pltpu.VMEM((HIST, BBLK), jnp.int32),
        pltpu.VMEM((BBLK, PADW), jnp.float32),
        pltpu.VMEM((BBLK, PADW), jnp.float32),
        pltpu.VMEM((DIM, BBLK), jnp.float32),
        pltpu.VMEM((DIM, BBLK), jnp.float32),
        pltpu.SemaphoreType.DMA,
        pltpu.SemaphoreType.DMA,
        pltpu.SemaphoreType.DMA,
        pltpu.SemaphoreType.DMA,
        pltpu.SemaphoreType.DMA,
    ],
    compiler_params=pltpu.CompilerParams(
        skip_device_barrier=True, needs_layout_passes=False
    ),
)
def _emb_lookup(
    table_hbm, idxt_hbm, out_hbm,
    idx_v, stg0, stg1, ob0, ob1, gs0, gs1, ws0, ws1, isem,
):
    c = lax.axis_index("c")
    s = lax.axis_index("s")
    w = s * NC + c

    pltpu.async_copy(idxt_hbm.at[:, w], idx_v, isem).wait()

    stgs = (stg0, stg1)
    obs = (ob0, ob1)
    gsems = (gs0, gs1)
    wsems = (ws0, ws1)

    rows = [lax.iota(jnp.int32, L) + bsub * L for bsub in range(BBLK // L)]

    def fire_gather(h, p):
        pltpu.async_copy(table_hbm.at[idx_v.at[h]], stgs[p], gsems[p])

    def wait_gather(p):
        pltpu.make_async_copy(table_hbm.at[pl.ds(0, BBLK)], stgs[p], gsems[p]).wait()

    def fire_write(h, p):
        pltpu.async_copy(obs[p], out_hbm.at[h, :, pl.ds(w * BBLK, BBLK)], wsems[p])

    def wait_write(p):
        pltpu.make_async_copy(
            out_hbm.at[0, :, pl.ds(0, BBLK)], obs[p], wsems[p]
        ).wait()

    def transpose(p):
        stg = stgs[p]
        ob = obs[p]

        def dbody(d, carry):
            colv = jnp.full((L,), d, dtype=jnp.int32)
            for bsub in range(BBLK // L):
                v = plsc.load_gather(stg, [rows[bsub], colv])
                ob[d, pl.ds(bsub * L, L)] = v
            return carry

        lax.fori_loop(0, DIM, dbody, None)

    fire_gather(0, 0)

    def hbody(hp, carry):
        h0 = hp * 2
        wait_gather(0)
        fire_gather(h0 + 1, 1)
        pl.when(hp > 0)(lambda: wait_write(0))
        transpose(0)
        fire_write(h0, 0)

        wait_gather(1)
        pl.when(h0 + 2 < HIST)(lambda: fire_gather(h0 + 2, 0))
        pl.when(hp > 0)(lambda: wait_write(1))
        transpose(1)
        fire_write(h0 + 1, 1)
        return carry

    lax.fori_loop(0, HIST // 2, hbody, None)
    wait_write(0)
    wait_write(1)


def kernel(inputs, emb_weight):
    table = _transpose_pad(emb_weight.T)
    idxt = inputs.T.reshape(HIST, NW, BBLK)
    out = _emb_lookup(table, idxt)
    return out.transpose(2, 0, 1)
